# baseline (speedup 1.0000x reference)
import functools

import jax
import jax.numpy as jnp
from jax import lax
from jax.experimental import pallas as pl
from jax.experimental.pallas import tpu as pltpu
from jax.experimental.pallas import tpu_sc as plsc

B = 4096
N = 8192
NC = 2
NS = 16
NW = NC * NS
ROWS_PER_W = B // NW
K = 8
GROUPS = ROWS_PER_W // K
LANES = 16
CHUNK = 2048
SUBS = N // CHUNK
NV = CHUNK // LANES


def _masked_cumsum_body(x_hbm, m_hbm, out_hbm, xb, mb, ob,
                        sx0, sx1, sm0, sm1, so0, so1):
    wid = lax.axis_index("s") * NC + lax.axis_index("c")
    base = wid * ROWS_PER_W
    sx = (sx0, sx1)
    sm = (sm0, sm1)
    so = (so0, so1)

    def idx(g, h):
        return (pl.ds(base + g * K, K), pl.ds(h * CHUNK, CHUNK))

    def x_copy(g, h):
        return pltpu.make_async_copy(x_hbm.at[idx(g, h)], xb.at[h % 2],
                                     sx[h % 2])

    def m_copy(g, h):
        return pltpu.make_async_copy(m_hbm.at[idx(g, h)], mb.at[h % 2],
                                     sm[h % 2])

    def o_copy(g, h):
        return pltpu.make_async_copy(ob.at[h % 2], out_hbm.at[idx(g, h)],
                                     so[h % 2])

    for h in range(2):
        x_copy(0, h).start()
        m_copy(0, h).start()

    def group(g, _):
        carries = tuple(jnp.float32(0.0) for _ in range(K))
        for h in range(SUBS):
            x_copy(g, h).wait()
            m_copy(g, h).wait()

            if h >= 2:
                o_copy(g, h - 2).wait()
            else:
                @pl.when(g > 0)
                def _(h=h):
                    o_copy(g - 1, h + 2).wait()

            def body(i, cs, h=h):
                p = h % 2
                col = pl.ds(i * LANES, LANES)
                new = []
                for k in range(K):
                    xm = xb[p, k, col] * mb[p, k, col]
                    s = plsc.cumsum(xm)
                    ob[p, k, col] = s + cs[k]
                    new.append(cs[k] + jnp.sum(xm))
                return tuple(new)

            carries = lax.fori_loop(0, NV, body, carries)
            o_copy(g, h).start()

            if h < 2:
                x_copy(g, h + 2).start()
                m_copy(g, h + 2).start()
            else:
                @pl.when(g + 1 < GROUPS)
                def _(h=h):
                    x_copy(g + 1, h - 2).start()
                    m_copy(g + 1, h - 2).start()
        return 0

    lax.fori_loop(0, GROUPS, group, 0)
    for h in range(2, 4):
        o_copy(GROUPS - 1, h).wait()


_mesh = plsc.VectorSubcoreMesh(core_axis_name="c", subcore_axis_name="s")

_masked_cumsum = functools.partial(
    pl.kernel,
    out_type=jax.ShapeDtypeStruct((B, N), jnp.float32),
    mesh=_mesh,
    compiler_params=pltpu.CompilerParams(needs_layout_passes=False),
    scratch_types=[
        pltpu.VMEM((2, K, CHUNK), jnp.float32),
        pltpu.VMEM((2, K, CHUNK), jnp.float32),
        pltpu.VMEM((2, K, CHUNK), jnp.float32),
        pltpu.SemaphoreType.DMA,
        pltpu.SemaphoreType.DMA,
        pltpu.SemaphoreType.DMA,
        pltpu.SemaphoreType.DMA,
        pltpu.SemaphoreType.DMA,
        pltpu.SemaphoreType.DMA,
    ],
)(_masked_cumsum_body)


def kernel(x, mask):
    return _masked_cumsum(x, mask.astype(jnp.float32))


# speedup vs baseline: 1.7258x; 1.7258x over previous
import functools

import jax
import jax.numpy as jnp
from jax import lax
from jax.experimental import pallas as pl
from jax.experimental.pallas import tpu as pltpu
from jax.experimental.pallas import tpu_sc as plsc

B = 4096
N = 8192
NC = 2
NS = 16
NW = NC * NS
ROWS_PER_W = B // NW
K = 4
GROUPS = ROWS_PER_W // K
LANES = 16
HALF = N // 2
NV = HALF // LANES


def _masked_cumsum_body(x_hbm, m_hbm, out_hbm, xb, mb, ob,
                        sx0, sx1, sm0, sm1, so0, so1):
    wid = lax.axis_index("s") * NC + lax.axis_index("c")
    base = wid * ROWS_PER_W
    sx = (sx0, sx1)
    sm = (sm0, sm1)
    so = (so0, so1)

    def idx(g, h):
        return (pl.ds(base + g * K, K), pl.ds(h * HALF, HALF))

    def x_copy(g, h):
        return pltpu.make_async_copy(x_hbm.at[idx(g, h)], xb.at[h], sx[h])

    def m_copy(g, h):
        return pltpu.make_async_copy(m_hbm.at[idx(g, h)], mb.at[h], sm[h])

    def o_copy(g, h):
        return pltpu.make_async_copy(ob.at[h], out_hbm.at[idx(g, h)], so[h])

    x_copy(0, 0).start()
    m_copy(0, 0).start()

    def group(g, _):
        carries = tuple(jnp.float32(0.0) for _ in range(K))
        for h in range(2):
            x_copy(g, h).wait()
            m_copy(g, h).wait()
            if h == 0:
                x_copy(g, 1).start()
                m_copy(g, 1).start()
            else:
                @pl.when(g + 1 < GROUPS)
                def _():
                    x_copy(g + 1, 0).start()
                    m_copy(g + 1, 0).start()

            @pl.when(g > 0)
            def _(h=h):
                o_copy(g - 1, h).wait()

            @plsc.parallel_loop(0, NV, carry=carries)
            def carries(i, cs, h=h):
                col = pl.ds(i * LANES, LANES)
                new = []
                for k in range(K):
                    xm = xb[h, k, col] * mb[h, k, col]
                    s = plsc.cumsum(xm)
                    ob[h, k, col] = s + cs[k]
                    new.append(cs[k] + jnp.sum(xm))
                return tuple(new)
            o_copy(g, h).start()
        return 0

    lax.fori_loop(0, GROUPS, group, 0)
    for h in range(2):
        o_copy(GROUPS - 1, h).wait()


_mesh = plsc.VectorSubcoreMesh(core_axis_name="c", subcore_axis_name="s")

_masked_cumsum = functools.partial(
    pl.kernel,
    out_type=jax.ShapeDtypeStruct((B, N), jnp.float32),
    mesh=_mesh,
    compiler_params=pltpu.CompilerParams(needs_layout_passes=False),
    scratch_types=[
        pltpu.VMEM((2, K, HALF), jnp.float32),
        pltpu.VMEM((2, K, HALF), jnp.float32),
        pltpu.VMEM((2, K, HALF), jnp.float32),
        pltpu.SemaphoreType.DMA,
        pltpu.SemaphoreType.DMA,
        pltpu.SemaphoreType.DMA,
        pltpu.SemaphoreType.DMA,
        pltpu.SemaphoreType.DMA,
        pltpu.SemaphoreType.DMA,
    ],
)(_masked_cumsum_body)


def kernel(x, mask):
    return _masked_cumsum(x, mask.astype(jnp.float32))
